# baseline (device time: 10557 ns/iter reference)
import jax
import jax.numpy as jnp
from jax import lax
from jax.experimental import pallas as pl
from jax.experimental.pallas import tpu as pltpu

CHUNKS = (2048, 1024, 1024)
NC = len(CHUNKS)
OFFS = tuple(sum(CHUNKS[:i]) for i in range(NC))


def kernel(x, W, labels):
    T, D = x.shape
    V = W.shape[1]
    labels2 = labels.reshape(1, T)

    def body(x_hbm, w_hbm, lab_hbm, out_hbm,
             x_v, w0_v, w1_v, w2_v, lab_v, out_v, send_buf, recv_buf,
             w_sems, misc_sems, send_sems, recv_sems):
        my_x = lax.axis_index("x")
        my_y = lax.axis_index("y")
        my_z = lax.axis_index("z")
        partner = (1 - my_x, my_y, my_z)
        w_bufs = [w0_v, w1_v, w2_v]

        cx = pltpu.make_async_copy(x_hbm, x_v, misc_sems.at[0])
        cx.start()
        cl = pltpu.make_async_copy(lab_hbm, lab_v, misc_sems.at[1])
        cl.start()
        w_copies = []
        for i in range(NC):
            c = pltpu.make_async_copy(
                w_hbm.at[:, pl.ds(OFFS[i], CHUNKS[i])], w_bufs[i],
                w_sems.at[i])
            c.start()
            w_copies.append(c)

        cx.wait()
        cl.wait()
        xb = x_v[:, :].astype(jnp.bfloat16)
        labrel = lab_v[:, :] - my_x * V

        def part(i):
            bv = CHUNKS[i]
            logits = lax.dot_general(
                w_bufs[i][:, :].astype(jnp.bfloat16), xb,
                dimension_numbers=(((0,), (1,)), ((), ())),
                preferred_element_type=jnp.float32,
            )
            s = jnp.sum(jnp.exp(logits), axis=0, keepdims=True)
            row = lax.broadcasted_iota(jnp.int32, (bv, T), 0)
            mask = row == (labrel - OFFS[i])
            c = jnp.sum(jnp.where(mask, logits, 0.0), axis=0, keepdims=True)
            return s, c

        def flush(slot, s, c):
            send_buf[slot, 0:1, :] = s
            send_buf[slot, 1:2, :] = c
            r = pltpu.make_async_remote_copy(
                src_ref=send_buf.at[slot],
                dst_ref=recv_buf.at[slot],
                send_sem=send_sems.at[slot],
                recv_sem=recv_sems.at[slot],
                device_id=partner,
                device_id_type=pl.DeviceIdType.MESH,
            )
            r.start()
            return r

        w_copies[0].wait()
        s0, c0 = part(0)

        barrier = pltpu.get_barrier_semaphore()
        pl.semaphore_signal(barrier, inc=1, device_id=partner,
                            device_id_type=pl.DeviceIdType.MESH)
        pl.semaphore_wait(barrier, 1)

        rdma0 = flush(0, s0, c0)

        w_copies[1].wait()
        s1, c1 = part(1)
        w_copies[2].wait()
        s2, c2 = part(2)
        rdma1 = flush(1, s1 + s2, c1 + c2)

        rdma0.wait()
        rdma1.wait()
        s_tot = s0 + s1 + s2 + recv_buf[0, 0:1, :] + recv_buf[1, 0:1, :]
        c_tot = c0 + c1 + c2 + recv_buf[0, 1:2, :] + recv_buf[1, 1:2, :]
        out_v[:, :] = jnp.log(s_tot) - c_tot
        co = pltpu.make_async_copy(out_v, out_hbm, misc_sems.at[2])
        co.start()
        co.wait()

    hbm = pltpu.MemorySpace.HBM
    out = pl.pallas_call(
        body,
        out_shape=jax.ShapeDtypeStruct((1, T), jnp.float32),
        in_specs=[
            pl.BlockSpec(memory_space=hbm),
            pl.BlockSpec(memory_space=hbm),
            pl.BlockSpec(memory_space=hbm),
        ],
        out_specs=pl.BlockSpec(memory_space=hbm),
        scratch_shapes=[
            pltpu.VMEM((T, D), jnp.float32),
            pltpu.VMEM((D, CHUNKS[0]), jnp.float32),
            pltpu.VMEM((D, CHUNKS[1]), jnp.float32),
            pltpu.VMEM((D, CHUNKS[2]), jnp.float32),
            pltpu.VMEM((1, T), jnp.int32),
            pltpu.VMEM((1, T), jnp.float32),
            pltpu.VMEM((2, 2, T), jnp.float32),
            pltpu.VMEM((2, 2, T), jnp.float32),
            pltpu.SemaphoreType.DMA((NC,)),
            pltpu.SemaphoreType.DMA((3,)),
            pltpu.SemaphoreType.DMA((2,)),
            pltpu.SemaphoreType.DMA((2,)),
        ],
        compiler_params=pltpu.CompilerParams(collective_id=0),
    )(
        pltpu.with_memory_space_constraint(x, hbm),
        pltpu.with_memory_space_constraint(W, hbm),
        pltpu.with_memory_space_constraint(labels2, hbm),
    )
    return out.reshape(T)


# device time: 10137 ns/iter; 1.0414x vs baseline; 1.0414x over previous
import jax
import jax.numpy as jnp
from jax import lax
from jax.experimental import pallas as pl
from jax.experimental.pallas import tpu as pltpu

CHUNKS = (3072, 1024)
NC = len(CHUNKS)
OFFS = tuple(sum(CHUNKS[:i]) for i in range(NC))


def kernel(x, W, labels):
    T, D = x.shape
    V = W.shape[1]
    labels2 = labels.reshape(1, T)

    def body(x_hbm, w_hbm, lab_hbm, out_hbm,
             x_v, w0_v, w1_v, lab_v, out_v, send_buf, recv_buf,
             w_sems, misc_sems, send_sems, recv_sems):
        my_x = lax.axis_index("x")
        my_y = lax.axis_index("y")
        my_z = lax.axis_index("z")
        partner = (1 - my_x, my_y, my_z)
        w_bufs = [w0_v, w1_v]

        cx = pltpu.make_async_copy(x_hbm, x_v, misc_sems.at[0])
        cx.start()
        cl = pltpu.make_async_copy(lab_hbm, lab_v, misc_sems.at[1])
        cl.start()
        w_copies = []
        for i in range(NC):
            c = pltpu.make_async_copy(
                w_hbm.at[:, pl.ds(OFFS[i], CHUNKS[i])], w_bufs[i],
                w_sems.at[i])
            c.start()
            w_copies.append(c)

        cx.wait()
        cl.wait()
        xb = x_v[:, :].astype(jnp.bfloat16)
        labrel = lab_v[:, :] - my_x * V

        def part(i):
            bv = CHUNKS[i]
            logits = lax.dot_general(
                w_bufs[i][:, :].astype(jnp.bfloat16), xb,
                dimension_numbers=(((0,), (1,)), ((), ())),
                preferred_element_type=jnp.float32,
            )
            s = jnp.sum(jnp.exp(logits), axis=0, keepdims=True)
            row = lax.broadcasted_iota(jnp.int32, (bv, T), 0)
            mask = row == (labrel - OFFS[i])
            c = jnp.sum(jnp.where(mask, logits, 0.0), axis=0, keepdims=True)
            return s, c

        def flush(slot, s, c):
            send_buf[slot, 0:1, :] = s
            send_buf[slot, 1:2, :] = c
            r = pltpu.make_async_remote_copy(
                src_ref=send_buf.at[slot],
                dst_ref=recv_buf.at[slot],
                send_sem=send_sems.at[slot],
                recv_sem=recv_sems.at[slot],
                device_id=partner,
                device_id_type=pl.DeviceIdType.MESH,
            )
            r.start()
            return r

        w_copies[0].wait()
        s0, c0 = part(0)

        barrier = pltpu.get_barrier_semaphore()
        pl.semaphore_signal(barrier, inc=1, device_id=partner,
                            device_id_type=pl.DeviceIdType.MESH)
        pl.semaphore_wait(barrier, 1)

        rdma0 = flush(0, s0, c0)

        w_copies[1].wait()
        s1, c1 = part(1)
        rdma1 = flush(1, s1, c1)

        rdma0.wait()
        rdma1.wait()
        s_tot = s0 + s1 + recv_buf[0, 0:1, :] + recv_buf[1, 0:1, :]
        c_tot = c0 + c1 + recv_buf[0, 1:2, :] + recv_buf[1, 1:2, :]
        out_v[:, :] = jnp.log(s_tot) - c_tot
        co = pltpu.make_async_copy(out_v, out_hbm, misc_sems.at[2])
        co.start()
        co.wait()

    hbm = pltpu.MemorySpace.HBM
    out = pl.pallas_call(
        body,
        out_shape=jax.ShapeDtypeStruct((1, T), jnp.float32),
        in_specs=[
            pl.BlockSpec(memory_space=hbm),
            pl.BlockSpec(memory_space=hbm),
            pl.BlockSpec(memory_space=hbm),
        ],
        out_specs=pl.BlockSpec(memory_space=hbm),
        scratch_shapes=[
            pltpu.VMEM((T, D), jnp.float32),
            pltpu.VMEM((D, CHUNKS[0]), jnp.float32),
            pltpu.VMEM((D, CHUNKS[1]), jnp.float32),
            pltpu.VMEM((1, T), jnp.int32),
            pltpu.VMEM((1, T), jnp.float32),
            pltpu.VMEM((2, 2, T), jnp.float32),
            pltpu.VMEM((2, 2, T), jnp.float32),
            pltpu.SemaphoreType.DMA((NC,)),
            pltpu.SemaphoreType.DMA((3,)),
            pltpu.SemaphoreType.DMA((2,)),
            pltpu.SemaphoreType.DMA((2,)),
        ],
        compiler_params=pltpu.CompilerParams(collective_id=0),
    )(
        pltpu.with_memory_space_constraint(x, hbm),
        pltpu.with_memory_space_constraint(W, hbm),
        pltpu.with_memory_space_constraint(labels2, hbm),
    )
    return out.reshape(T)


# device time: 9907 ns/iter; 1.0656x vs baseline; 1.0232x over previous
import jax
import jax.numpy as jnp
from jax import lax
from jax.experimental import pallas as pl
from jax.experimental.pallas import tpu as pltpu

CHUNKS = (2048, 2048)
NC = len(CHUNKS)
OFFS = tuple(sum(CHUNKS[:i]) for i in range(NC))


def kernel(x, W, labels):
    T, D = x.shape
    V = W.shape[1]
    labels2 = labels.reshape(1, T)

    def body(x_hbm, w_hbm, lab_hbm, out_hbm,
             x_v, w0_v, w1_v, lab_v, out_v, send_buf, recv_buf,
             w_sems, misc_sems, send_sems, recv_sems):
        my_x = lax.axis_index("x")
        my_y = lax.axis_index("y")
        my_z = lax.axis_index("z")
        partner = (1 - my_x, my_y, my_z)
        w_bufs = [w0_v, w1_v]

        cx = pltpu.make_async_copy(x_hbm, x_v, misc_sems.at[0])
        cx.start()
        cl = pltpu.make_async_copy(lab_hbm, lab_v, misc_sems.at[1])
        cl.start()
        w_copies = []
        for i in range(NC):
            c = pltpu.make_async_copy(
                w_hbm.at[:, pl.ds(OFFS[i], CHUNKS[i])], w_bufs[i],
                w_sems.at[i])
            c.start()
            w_copies.append(c)

        cx.wait()
        cl.wait()
        xb = x_v[:, :].astype(jnp.bfloat16)
        labrel = lab_v[:, :] - my_x * V

        def part(i):
            bv = CHUNKS[i]
            logits = lax.dot_general(
                w_bufs[i][:, :].astype(jnp.bfloat16), xb,
                dimension_numbers=(((0,), (1,)), ((), ())),
                preferred_element_type=jnp.float32,
            )
            s = jnp.sum(jnp.exp(logits), axis=0, keepdims=True)
            row = lax.broadcasted_iota(jnp.int32, (bv, T), 0)
            mask = row == (labrel - OFFS[i])
            c = jnp.sum(jnp.where(mask, logits, 0.0), axis=0, keepdims=True)
            return s, c

        def flush(slot, s, c):
            send_buf[slot, 0:1, :] = s
            send_buf[slot, 1:2, :] = c
            r = pltpu.make_async_remote_copy(
                src_ref=send_buf.at[slot],
                dst_ref=recv_buf.at[slot],
                send_sem=send_sems.at[slot],
                recv_sem=recv_sems.at[slot],
                device_id=partner,
                device_id_type=pl.DeviceIdType.MESH,
            )
            r.start()
            return r

        w_copies[0].wait()
        s0, c0 = part(0)

        barrier = pltpu.get_barrier_semaphore()
        pl.semaphore_signal(barrier, inc=1, device_id=partner,
                            device_id_type=pl.DeviceIdType.MESH)
        pl.semaphore_wait(barrier, 1)

        rdma0 = flush(0, s0, c0)

        w_copies[1].wait()
        s1, c1 = part(1)
        rdma1 = flush(1, s1, c1)

        rdma0.wait()
        rdma1.wait()
        s_tot = s0 + s1 + recv_buf[0, 0:1, :] + recv_buf[1, 0:1, :]
        c_tot = c0 + c1 + recv_buf[0, 1:2, :] + recv_buf[1, 1:2, :]
        out_v[:, :] = jnp.log(s_tot) - c_tot
        co = pltpu.make_async_copy(out_v, out_hbm, misc_sems.at[2])
        co.start()
        co.wait()

    hbm = pltpu.MemorySpace.HBM
    out = pl.pallas_call(
        body,
        out_shape=jax.ShapeDtypeStruct((1, T), jnp.float32),
        in_specs=[
            pl.BlockSpec(memory_space=hbm),
            pl.BlockSpec(memory_space=hbm),
            pl.BlockSpec(memory_space=hbm),
        ],
        out_specs=pl.BlockSpec(memory_space=hbm),
        scratch_shapes=[
            pltpu.VMEM((T, D), jnp.float32),
            pltpu.VMEM((D, CHUNKS[0]), jnp.float32),
            pltpu.VMEM((D, CHUNKS[1]), jnp.float32),
            pltpu.VMEM((1, T), jnp.int32),
            pltpu.VMEM((1, T), jnp.float32),
            pltpu.VMEM((2, 2, T), jnp.float32),
            pltpu.VMEM((2, 2, T), jnp.float32),
            pltpu.SemaphoreType.DMA((NC,)),
            pltpu.SemaphoreType.DMA((3,)),
            pltpu.SemaphoreType.DMA((2,)),
            pltpu.SemaphoreType.DMA((2,)),
        ],
        compiler_params=pltpu.CompilerParams(collective_id=0),
    )(
        pltpu.with_memory_space_constraint(x, hbm),
        pltpu.with_memory_space_constraint(W, hbm),
        pltpu.with_memory_space_constraint(labels2, hbm),
    )
    return out.reshape(T)
